# Initial kernel scaffold; baseline (speedup 1.0000x reference)
import jax
import jax.numpy as jnp
from jax.experimental import pallas as pl


def kernel(x, edge_index, W1, b1, W2, b2):
    raise NotImplementedError("write your pallas kernel here")


# speedup vs baseline: 52.4366x; 52.4366x over previous
import jax
import jax.numpy as jnp
from jax import lax
from jax.experimental import pallas as pl
from jax.experimental.pallas import tpu as pltpu
from jax.experimental.pallas import tpu_sc as plsc

N = 10000
NP = 10112
F = 16
NC, NS = 2, 16
ROWS = NP // NS
B = 1024
Q = 10
EPT = Q * B
EPAD = NC * NS * EPT


def _mesh():
    return plsc.VectorSubcoreMesh(
        core_axis_name="c", subcore_axis_name="s",
        num_cores=NC, num_subcores=NS)


_SC_PARAMS = pltpu.CompilerParams(use_tc_tiling_on_sc=False)


def _sc_deg_body(dst_hbm, z_hbm, ones_hbm, out_hbm, idx_d, ones_v, acc):
    c = lax.axis_index("c")
    s = lax.axis_index("s")
    sl = pl.ds(s * ROWS, ROWS)
    pltpu.sync_copy(z_hbm.at[sl], acc.at[sl])
    pltpu.sync_copy(ones_hbm, ones_v)
    pltpu.sync_copy(dst_hbm.at[c, s], idx_d)
    plsc.subcore_barrier()

    def step(q, carry):
        pltpu.sync_copy(ones_v, acc.at[idx_d.at[q]], add=True)
        return carry

    lax.fori_loop(0, Q, step, 0)
    plsc.subcore_barrier()
    pltpu.sync_copy(acc.at[sl], out_hbm.at[c, sl])


def _sc_deg(dsts, zeros, ones):
    return pl.kernel(
        _sc_deg_body,
        out_type=jax.ShapeDtypeStruct((NC, NP, F), jnp.float32),
        mesh=_mesh(),
        scratch_types=[
            pltpu.VMEM((Q, B), jnp.int32),
            pltpu.VMEM((B, F), jnp.float32),
            pltpu.VMEM_SHARED((NP, F), jnp.float32),
        ],
        compiler_params=_SC_PARAMS,
    )(dsts, zeros, ones)


def _sc_edge_body(y_hbm, src_hbm, dst_hbm, z_hbm, out_hbm,
                  idx_s, idx_d, rows, ysp, acc, gsem, ssem):
    c = lax.axis_index("c")
    s = lax.axis_index("s")
    sl = pl.ds(s * ROWS, ROWS)
    pltpu.sync_copy(z_hbm.at[sl], acc.at[sl])
    pltpu.sync_copy(y_hbm.at[sl], ysp.at[sl])
    pltpu.sync_copy(src_hbm.at[c, s], idx_s)
    pltpu.sync_copy(dst_hbm.at[c, s], idx_d)
    plsc.subcore_barrier()

    def gather(q, buf):
        return pltpu.async_copy(ysp.at[idx_s.at[q]], rows.at[buf], gsem)

    def scat(q, buf):
        return pltpu.async_copy(rows.at[buf], acc.at[idx_d.at[q]], ssem,
                                add=True)

    gather(0, 0).wait()

    def step(q, carry):
        buf = lax.rem(q, 2)
        nbuf = 1 - buf
        g = gather(q + 1, nbuf)
        sc = scat(q, buf)
        sc.wait()
        g.wait()
        return carry

    lax.fori_loop(0, Q - 1, step, 0)
    scat(Q - 1, lax.rem(Q - 1, 2)).wait()
    plsc.subcore_barrier()
    pltpu.sync_copy(acc.at[sl], out_hbm.at[c, sl])


def _sc_edge(y, srcs, dsts, zeros):
    return pl.kernel(
        _sc_edge_body,
        out_type=jax.ShapeDtypeStruct((NC, NP, F), jnp.float32),
        mesh=_mesh(),
        scratch_types=[
            pltpu.VMEM((Q, B), jnp.int32),
            pltpu.VMEM((Q, B), jnp.int32),
            pltpu.VMEM((2, B, F), jnp.float32),
            pltpu.VMEM_SHARED((NP, F), jnp.float32),
            pltpu.VMEM_SHARED((NP, F), jnp.float32),
            pltpu.SemaphoreType.DMA,
            pltpu.SemaphoreType.DMA,
        ],
        compiler_params=_SC_PARAMS,
    )(y, srcs, dsts, zeros)


def _tc_mm(x, w):
    def body(x_ref, w_ref, o_ref):
        o_ref[...] = jnp.dot(x_ref[...], w_ref[...],
                             preferred_element_type=jnp.float32)

    return pl.pallas_call(
        body,
        out_shape=jax.ShapeDtypeStruct((x.shape[0], w.shape[1]), jnp.float32),
    )(x, w)


def _tc_prep(degp, xw):
    def body(p_ref, xw_ref, y_ref, sl_ref, di_ref):
        deg = p_ref[0] + p_ref[1] + 1.0
        dinv = lax.rsqrt(deg)
        y = xw_ref[...] * dinv
        y_ref[...] = y
        sl_ref[...] = y * dinv
        di_ref[...] = dinv

    s = jax.ShapeDtypeStruct((NP, F), jnp.float32)
    return pl.pallas_call(body, out_shape=(s, s, s))(degp, xw)


def _tc_mid(part, sl1, dinv, b1, w2):
    def body(q_ref, sl_ref, di_ref, b_ref, w_ref, y_ref, s2_ref):
        di = di_ref[...]
        h = di * (q_ref[0] + q_ref[1]) + sl_ref[...] + b_ref[...]
        h = jnp.maximum(h, 0.0)
        xw2 = jnp.dot(h, w_ref[...], preferred_element_type=jnp.float32)
        y2 = xw2 * di
        y_ref[...] = y2
        s2_ref[...] = y2 * di

    s = jax.ShapeDtypeStruct((NP, F), jnp.float32)
    return pl.pallas_call(body, out_shape=(s, s))(part, sl1, dinv, b1, w2)


def _tc_fin(part, sl2, dinv, b2):
    def body(r_ref, s2_ref, di_ref, b_ref, o_ref):
        o_ref[...] = (di_ref[...] * (r_ref[0] + r_ref[1])
                      + s2_ref[...] + b_ref[...])

    s = jax.ShapeDtypeStruct((NP, F), jnp.float32)
    return pl.pallas_call(body, out_shape=s)(part, sl2, dinv, b2)


def kernel(x, edge_index, W1, b1, W2, b2):
    src = edge_index[0].astype(jnp.int32)
    dst = edge_index[1].astype(jnp.int32)
    e = src.shape[0]
    pad = jnp.full((EPAD - e,), N, jnp.int32)
    srcs = jnp.concatenate([src, pad]).reshape(NC, NS, Q, B)
    dsts = jnp.concatenate([dst, pad]).reshape(NC, NS, Q, B)
    xp = jnp.pad(x, ((0, NP - N), (0, 0)))
    zeros = jnp.zeros((NP, F), jnp.float32)
    ones = jnp.ones((B, F), jnp.float32)

    degp = _sc_deg(dsts, zeros, ones)
    xw1 = _tc_mm(xp, W1)
    y1, sl1, dinv = _tc_prep(degp, xw1)
    p1 = _sc_edge(y1, srcs, dsts, zeros)
    y2, sl2 = _tc_mid(p1, sl1, dinv, b1.reshape(1, F), W2)
    p2 = _sc_edge(y2, srcs, dsts, zeros)
    outp = _tc_fin(p2, sl2, dinv, b2.reshape(1, F))
    return outp[:N]


# speedup vs baseline: 86.5132x; 1.6499x over previous
import jax
import jax.numpy as jnp
from jax import lax
from jax.experimental import pallas as pl
from jax.experimental.pallas import tpu as pltpu
from jax.experimental.pallas import tpu_sc as plsc

N = 10000
NP = 10112
F = 16
NR = NP * F // 128
NC, NS = 2, 16
ROWS = NP // NS
B = 1000
Q = 10


def _mesh():
    return plsc.VectorSubcoreMesh(
        core_axis_name="c", subcore_axis_name="s",
        num_cores=NC, num_subcores=NS)


_SC_PARAMS = pltpu.CompilerParams(use_tc_tiling_on_sc=False)


def _sc_deg_body(dst_hbm, z_hbm, ones_hbm, out_hbm, idx_d, ones_v, acc):
    c = lax.axis_index("c")
    s = lax.axis_index("s")
    sl = pl.ds(s * ROWS, ROWS)
    pltpu.sync_copy(z_hbm.at[sl], acc.at[sl])
    pltpu.sync_copy(ones_hbm, ones_v)
    pltpu.sync_copy(dst_hbm.at[c, s], idx_d)
    plsc.subcore_barrier()

    def step(q, carry):
        pltpu.sync_copy(ones_v, acc.at[idx_d.at[q]], add=True)
        return carry

    lax.fori_loop(0, Q, step, 0)
    plsc.subcore_barrier()
    pltpu.sync_copy(acc.at[sl], out_hbm.at[c, sl])


def _sc_deg(dsts, zeros, ones):
    return pl.kernel(
        _sc_deg_body,
        out_type=jax.ShapeDtypeStruct((NC, NP, F), jnp.float32),
        mesh=_mesh(),
        scratch_types=[
            pltpu.VMEM((Q, B), jnp.int32),
            pltpu.VMEM((B, F), jnp.float32),
            pltpu.VMEM_SHARED((NP, F), jnp.float32),
        ],
        compiler_params=_SC_PARAMS,
    )(dsts, zeros, ones)


def _sc_edge_body(y_hbm, src_hbm, dst_hbm, z_hbm, out_hbm,
                  idx_s, idx_d, rows, ysp, acc, gsem, ssem):
    c = lax.axis_index("c")
    s = lax.axis_index("s")
    sl = pl.ds(s * ROWS, ROWS)
    pltpu.sync_copy(z_hbm.at[sl], acc.at[sl])
    pltpu.sync_copy(y_hbm.at[sl], ysp.at[sl])
    pltpu.sync_copy(src_hbm.at[c, s], idx_s)
    pltpu.sync_copy(dst_hbm.at[c, s], idx_d)
    plsc.subcore_barrier()

    def gather(q, buf):
        return pltpu.async_copy(ysp.at[idx_s.at[q]], rows.at[buf], gsem)

    def scat(q, buf):
        return pltpu.async_copy(rows.at[buf], acc.at[idx_d.at[q]], ssem,
                                add=True)

    gather(0, 0).wait()

    def step(q, carry):
        buf = lax.rem(q, 2)
        nbuf = 1 - buf
        g = gather(q + 1, nbuf)
        sc = scat(q, buf)
        sc.wait()
        g.wait()
        return carry

    lax.fori_loop(0, Q - 1, step, 0)
    scat(Q - 1, lax.rem(Q - 1, 2)).wait()
    plsc.subcore_barrier()
    pltpu.sync_copy(acc.at[sl], out_hbm.at[c, sl])


def _sc_edge(y, srcs, dsts, zeros):
    return pl.kernel(
        _sc_edge_body,
        out_type=jax.ShapeDtypeStruct((NC, NP, F), jnp.float32),
        mesh=_mesh(),
        scratch_types=[
            pltpu.VMEM((Q, B), jnp.int32),
            pltpu.VMEM((Q, B), jnp.int32),
            pltpu.VMEM((2, B, F), jnp.float32),
            pltpu.VMEM_SHARED((NP, F), jnp.float32),
            pltpu.VMEM_SHARED((NP, F), jnp.float32),
            pltpu.SemaphoreType.DMA,
            pltpu.SemaphoreType.DMA,
        ],
        compiler_params=_SC_PARAMS,
    )(y, srcs, dsts, zeros)


def _tc_mm(x, w):
    def body(x_ref, w_ref, o_ref):
        o_ref[...] = jnp.dot(x_ref[...], w_ref[...],
                             preferred_element_type=jnp.float32)

    return pl.pallas_call(
        body,
        out_shape=jax.ShapeDtypeStruct((x.shape[0], w.shape[1]), jnp.float32),
    )(x, w)


def _tc_prep(degp, xw):
    def body(p_ref, xw_ref, y_ref, di_ref):
        dinv = lax.rsqrt(p_ref[0] + p_ref[1] + 1.0)
        y_ref[...] = xw_ref[...] * dinv
        di_ref[...] = dinv

    s = jax.ShapeDtypeStruct((NR, 128), jnp.float32)
    return pl.pallas_call(body, out_shape=(s, s))(degp, xw)


def _tc_mid(part, dinv, xw, b1):
    def body(q_ref, di_ref, xw_ref, b_ref, g_ref):
        di = di_ref[...]
        h = di * (q_ref[0] + q_ref[1]) + xw_ref[...] * di * di + b_ref[...]
        g_ref[...] = jnp.maximum(h, 0.0) * di

    s = jax.ShapeDtypeStruct((NR, 128), jnp.float32)
    return pl.pallas_call(body, out_shape=s)(part, dinv, xw, b1)


def _tc_fin(part, dinv, g, w2big, b2):
    def body(r_ref, di_ref, g_ref, w_ref, b_ref, o_ref):
        di = di_ref[...]
        z = di * (r_ref[0] + r_ref[1]) + g_ref[...] * di
        o_ref[...] = jnp.dot(z, w_ref[...], precision=lax.Precision.HIGHEST,
                             preferred_element_type=jnp.float32) + b_ref[...]

    s = jax.ShapeDtypeStruct((NR, 128), jnp.float32)
    return pl.pallas_call(body, out_shape=s)(part, dinv, g, w2big, b2)


def kernel(x, edge_index, W1, b1, W2, b2):
    src = edge_index[0].astype(jnp.int32)
    dst = edge_index[1].astype(jnp.int32)
    srcs = src.reshape(NC, NS, Q, B)
    dsts = dst.reshape(NC, NS, Q, B)
    xp = jnp.pad(x, ((0, NP - N), (0, 0)))
    zeros = jnp.zeros((NP, F), jnp.float32)
    ones = jnp.ones((B, F), jnp.float32)
    b1r = jnp.tile(b1, 8).reshape(1, 128)
    b2r = jnp.tile(b2, 8).reshape(1, 128)
    w2big = jnp.kron(jnp.eye(8, dtype=jnp.float32), W2)

    degp = _sc_deg(dsts, zeros, ones)
    xw128 = _tc_mm(xp, W1).reshape(NR, 128)
    y1, dinv = _tc_prep(degp.reshape(NC, NR, 128), xw128)
    p1 = _sc_edge(y1.reshape(NP, F), srcs, dsts, zeros)
    g = _tc_mid(p1.reshape(NC, NR, 128), dinv, xw128, b1r)
    p2 = _sc_edge(g.reshape(NP, F), srcs, dsts, zeros)
    out = _tc_fin(p2.reshape(NC, NR, 128), dinv, g, w2big, b2r)
    return out.reshape(NP, F)[:N]
